# baseline (speedup 1.0000x reference)
import functools

import jax
import jax.numpy as jnp
from jax import lax
from jax.experimental import pallas as pl
from jax.experimental.pallas import tpu as pltpu
from jax.experimental.pallas import tpu_sc as plsc

D = 768
H = 256
O = 768
N = 8192
K = 32

ROWS_PER_BLK = 2048
NBLK = N // ROWS_PER_BLK

LANES = 16
CHUNK = 512
NSLICE = CHUNK // LANES
NSUB = 16
NEG = float("-inf")
BIG = 1 << 30
INV_SQRT_H = 1.0 / (H ** 0.5)


def _logits_body(ce, wq, bq, wk, e, out, vm, vz, w_s):
    i = pl.program_id(0)

    @pl.when(i == 0)
    def _():
        ce2 = ce[...].reshape(1, D)
        bq2 = bq[...].reshape(1, H)
        q = jnp.dot(ce2, wq[...], preferred_element_type=jnp.float32,
                    precision=lax.Precision.HIGHEST) + bq2
        w = lax.dot_general(q, wk[...], (((1,), (1,)), ((), ())),
                            preferred_element_type=jnp.float32,
                            precision=lax.Precision.HIGHEST)
        w_s[...] = w * INV_SQRT_H

    lb = lax.dot_general(w_s[...], e[...], (((1,), (1,)), ((), ())),
                         preferred_element_type=jnp.float32)
    out[...] = lb.reshape(ROWS_PER_BLK)

    @pl.when(i == 0)
    def _():
        vm[...] = lb
        vz[...] = jnp.ones((1, ROWS_PER_BLK), jnp.float32)

    @pl.when(i > 0)
    def _():
        mnew = jnp.maximum(vm[...], lb)
        vz[...] = vz[...] * jnp.exp(vm[...] - mnew) + jnp.exp(lb - mnew)
        vm[...] = mnew


def _logits_call(ce, Wq, bq, Wk, E):
    return pl.pallas_call(
        _logits_body,
        grid=(NBLK,),
        in_specs=[
            pl.BlockSpec((D,), lambda i: (0,)),
            pl.BlockSpec((D, H), lambda i: (0, 0)),
            pl.BlockSpec((H,), lambda i: (0,)),
            pl.BlockSpec((D, H), lambda i: (0, 0)),
            pl.BlockSpec((ROWS_PER_BLK, D), lambda i: (i, 0)),
        ],
        out_specs=[
            pl.BlockSpec((ROWS_PER_BLK,), lambda i: (i,)),
            pl.BlockSpec((1, ROWS_PER_BLK), lambda i: (0, 0)),
            pl.BlockSpec((1, ROWS_PER_BLK), lambda i: (0, 0)),
        ],
        out_shape=[
            jax.ShapeDtypeStruct((N,), jnp.float32),
            jax.ShapeDtypeStruct((1, ROWS_PER_BLK), jnp.float32),
            jax.ShapeDtypeStruct((1, ROWS_PER_BLK), jnp.float32),
        ],
        scratch_shapes=[pltpu.VMEM((1, D), jnp.float32)],
    )(ce, Wq, bq, Wk, E)


def _bmax(x, lane):
    for st in (1, 2, 4, 8):
        x = jnp.maximum(x, jnp.take_along_axis(x, lane ^ st, axis=0))
    return x


def _bmin(x, lane):
    for st in (1, 2, 4, 8):
        x = jnp.minimum(x, jnp.take_along_axis(x, lane ^ st, axis=0))
    return x


def _topk_body(lg_hbm, e_hbm, sel_out, val_out,
               lg_v, sv_sh, si_sh, cv_v, ci_v, lv_v, li_v, idx_v, val_v,
               idx1_v, row1_v, sem):
    c = lax.axis_index("c")
    s = lax.axis_index("s")
    lane = lax.iota(jnp.int32, 16)
    base = s * CHUNK

    pltpu.sync_copy(lg_hbm.at[pl.ds(base, CHUNK)], lg_v)

    def cm_body(k, mv):
        return jnp.maximum(mv, plsc.load_gather(lg_v, [k * 16 + lane]))
    mv = lax.fori_loop(0, NSLICE, cm_body,
                       jnp.full((16,), NEG, jnp.float32), unroll=4)

    def extract_one(j, mv):
        m = _bmax(mv, lane)
        ls = _bmin(jnp.where(mv == m, lane, BIG), lane)
        c0 = plsc.load_gather(lg_v, [lane * 16 + ls])
        c1 = plsc.load_gather(lg_v, [(lane + 16) * 16 + ls])
        r0 = jnp.where(c0 == m, lane, BIG)
        r1 = jnp.where(c1 == m, lane + 16, BIG)
        rs = _bmin(jnp.minimum(r0, r1), lane)
        gidx = rs * 16 + ls
        lane0 = lane == 0
        jv = jnp.full((16,), j, jnp.int32)
        plsc.store_scatter(lv_v, [jv], m, mask=lane0)
        plsc.store_scatter(li_v, [jv], gidx + base, mask=lane0)
        plsc.store_scatter(lg_v, [gidx],
                          jnp.full((16,), NEG, jnp.float32), mask=lane0)
        c0 = jnp.where(lane == rs, NEG, c0)
        c1 = jnp.where(lane + 16 == rs, NEG, c1)
        m2 = _bmax(jnp.maximum(c0, c1), lane)
        return jnp.where(lane == ls, m2, mv)

    lax.fori_loop(0, K, extract_one, mv, unroll=2)

    pltpu.sync_copy(lv_v, sv_sh.at[pl.ds(s * K, K)])
    pltpu.sync_copy(li_v, si_sh.at[pl.ds(s * K, K)])
    plsc.subcore_barrier()

    if True:
        pltpu.sync_copy(sv_sh, cv_v)
        pltpu.sync_copy(si_sh, ci_v)

        def merge_body(j, carry):
            ptr, resv0, resv1, resi0, resi1 = carry
            pc = lane * K + jnp.minimum(ptr, K - 1)
            hv = plsc.load_gather(cv_v, [pc])
            hi = plsc.load_gather(ci_v, [pc])
            hv = jnp.where(ptr > K - 1, NEG, hv)
            m = _bmax(hv, lane)
            cand = jnp.where(hv == m, hi, BIG)
            gi = _bmin(cand, lane)
            pick = jnp.logical_and(cand == gi, hv == m)
            ptr = ptr + jnp.where(pick, 1, 0)
            jm = lax.rem(j, 16)
            in0 = j < 16
            at = lane == jm
            resv0 = jnp.where(jnp.logical_and(in0, at), m, resv0)
            resi0 = jnp.where(jnp.logical_and(in0, at), gi, resi0)
            resv1 = jnp.where(jnp.logical_and(jnp.logical_not(in0), at), m, resv1)
            resi1 = jnp.where(jnp.logical_and(jnp.logical_not(in0), at), gi, resi1)
            return ptr, resv0, resv1, resi0, resi1

        init = (jnp.zeros((16,), jnp.int32),
                jnp.full((16,), NEG, jnp.float32),
                jnp.full((16,), NEG, jnp.float32),
                jnp.zeros((16,), jnp.int32),
                jnp.zeros((16,), jnp.int32))
        _, resv0, resv1, resi0, resi1 = lax.fori_loop(0, K, merge_body, init,
                                                      unroll=2)

        is0 = c == 0
        myv = jnp.where(is0, resv0, resv1)
        myi = jnp.where(is0, resi0, resi1)

        @pl.when(s == 0)
        def _():
            val_v[...] = myv
            pltpu.sync_copy(val_v, val_out.at[pl.ds(c * 16, 16)])

        mine = jnp.take_along_axis(myi, jnp.full((16,), s, jnp.int32), axis=0)
        plsc.store_scatter(idx1_v, [jnp.zeros((16,), jnp.int32)], mine,
                           mask=lane == 0)
        pltpu.async_copy(e_hbm.at[idx1_v], row1_v, sem).wait()
        pltpu.sync_copy(row1_v, sel_out.at[pl.ds(c * 16 + s, 1)])


_topk_gather = functools.partial(
    pl.kernel,
    out_type=(jax.ShapeDtypeStruct((K, D), jnp.float32),
              jax.ShapeDtypeStruct((K,), jnp.float32)),
    mesh=plsc.VectorSubcoreMesh(core_axis_name="c", subcore_axis_name="s"),
    compiler_params=pltpu.CompilerParams(needs_layout_passes=False),
    scratch_types=[
        pltpu.VMEM((CHUNK,), jnp.float32),
        pltpu.VMEM_SHARED((NSUB * K,), jnp.float32),
        pltpu.VMEM_SHARED((NSUB * K,), jnp.int32),
        pltpu.VMEM((NSUB * K,), jnp.float32),
        pltpu.VMEM((NSUB * K,), jnp.int32),
        pltpu.VMEM((K,), jnp.float32),
        pltpu.VMEM((K,), jnp.int32),
        pltpu.VMEM((16,), jnp.int32),
        pltpu.VMEM((16,), jnp.float32),
        pltpu.VMEM((1,), jnp.int32),
        pltpu.VMEM((1, D), jnp.float32),
        pltpu.SemaphoreType.DMA,
    ],
)(_topk_body)


def _finish_body(vm, vz, topv, sel, wv, bv, wo, bo, out):
    m = jnp.max(vm[...])
    z = jnp.sum(vz[...] * jnp.exp(vm[...] - m))
    s = jnp.exp(topv[...].reshape(1, K) - m) / z
    ssum = jnp.sum(s)
    ws = jnp.dot(s, sel[...], preferred_element_type=jnp.float32)
    h = (jnp.dot(ws, wv[...], preferred_element_type=jnp.float32)
         + ssum * bv[...].reshape(1, H))
    o = jnp.dot(h, wo[...], preferred_element_type=jnp.float32)
    out[...] = (o + bo[...].reshape(1, O)).reshape(O)


def _finish_call(vm, vz, topv, sel, Wv, bv, Wo, bo):
    return pl.pallas_call(
        _finish_body,
        out_shape=jax.ShapeDtypeStruct((O,), jnp.float32),
    )(vm, vz, topv, sel, Wv, bv, Wo, bo)


def kernel(class_embedding, entity_embeddings, Wq, bq, Wk, bk, Wv, bv, Wo, bo):
    del bk
    lg, vm, vz = _logits_call(class_embedding, Wq, bq, Wk, entity_embeddings)
    sel, topv = _topk_gather(lg, entity_embeddings)
    return _finish_call(vm, vz, topv, sel, Wv, bv, Wo, bo)


# speedup vs baseline: 1.0035x; 1.0035x over previous
import functools

import jax
import jax.numpy as jnp
from jax import lax
from jax.experimental import pallas as pl
from jax.experimental.pallas import tpu as pltpu
from jax.experimental.pallas import tpu_sc as plsc

D = 768
H = 256
O = 768
N = 8192
K = 32

ROWS_PER_BLK = 2048
NBLK = N // ROWS_PER_BLK

LANES = 16
CHUNK = 512
NSLICE = CHUNK // LANES
NSUB = 16
NEG = float("-inf")
BIG = 1 << 30
INV_SQRT_H = 1.0 / (H ** 0.5)


def _logits_body(ce, wq, bq, wk, e, out, vm, vz, w_s):
    i = pl.program_id(0)

    @pl.when(i == 0)
    def _():
        ce2 = ce[...].reshape(1, D)
        bq2 = bq[...].reshape(1, H)
        q = jnp.dot(ce2, wq[...], preferred_element_type=jnp.float32,
                    precision=lax.Precision.HIGHEST) + bq2
        w = lax.dot_general(q, wk[...], (((1,), (1,)), ((), ())),
                            preferred_element_type=jnp.float32,
                            precision=lax.Precision.HIGHEST)
        w_s[...] = w * INV_SQRT_H

    lb = lax.dot_general(w_s[...], e[...], (((1,), (1,)), ((), ())),
                         preferred_element_type=jnp.float32)
    out[...] = lb.reshape(ROWS_PER_BLK)

    @pl.when(i == 0)
    def _():
        vm[...] = lb
        vz[...] = jnp.ones((1, ROWS_PER_BLK), jnp.float32)

    @pl.when(i > 0)
    def _():
        mnew = jnp.maximum(vm[...], lb)
        vz[...] = vz[...] * jnp.exp(vm[...] - mnew) + jnp.exp(lb - mnew)
        vm[...] = mnew


def _logits_call(ce, Wq, bq, Wk, E):
    return pl.pallas_call(
        _logits_body,
        grid=(NBLK,),
        in_specs=[
            pl.BlockSpec((D,), lambda i: (0,)),
            pl.BlockSpec((D, H), lambda i: (0, 0)),
            pl.BlockSpec((H,), lambda i: (0,)),
            pl.BlockSpec((D, H), lambda i: (0, 0)),
            pl.BlockSpec((ROWS_PER_BLK, D), lambda i: (i, 0)),
        ],
        out_specs=[
            pl.BlockSpec((ROWS_PER_BLK,), lambda i: (i,)),
            pl.BlockSpec((1, ROWS_PER_BLK), lambda i: (0, 0)),
            pl.BlockSpec((1, ROWS_PER_BLK), lambda i: (0, 0)),
        ],
        out_shape=[
            jax.ShapeDtypeStruct((N,), jnp.float32),
            jax.ShapeDtypeStruct((1, ROWS_PER_BLK), jnp.float32),
            jax.ShapeDtypeStruct((1, ROWS_PER_BLK), jnp.float32),
        ],
        scratch_shapes=[pltpu.VMEM((1, D), jnp.float32)],
    )(ce, Wq, bq, Wk, E)


def _bmax(x, lane):
    for st in (1, 2, 4, 8):
        x = jnp.maximum(x, jnp.take_along_axis(x, lane ^ st, axis=0))
    return x


def _bmin(x, lane):
    for st in (1, 2, 4, 8):
        x = jnp.minimum(x, jnp.take_along_axis(x, lane ^ st, axis=0))
    return x


def _topk_body(lg_hbm, e_hbm, sel_out, val_out,
               lg_v, sv_sh, si_sh, cv_v, ci_v, lv_v, li_v, idx_v, val_v,
               idx1_v, row1_v, sem):
    c = lax.axis_index("c")
    s = lax.axis_index("s")
    lane = lax.iota(jnp.int32, 16)
    base = s * CHUNK

    pltpu.sync_copy(lg_hbm.at[pl.ds(base, CHUNK)], lg_v)

    def cm_body(k, mv):
        return jnp.maximum(mv, plsc.load_gather(lg_v, [k * 16 + lane]))
    mv = lax.fori_loop(0, NSLICE, cm_body,
                       jnp.full((16,), NEG, jnp.float32), unroll=4)

    def extract_one(j, mv):
        m = _bmax(mv, lane)
        ls = _bmin(jnp.where(mv == m, lane, BIG), lane)
        c0 = plsc.load_gather(lg_v, [lane * 16 + ls])
        c1 = plsc.load_gather(lg_v, [(lane + 16) * 16 + ls])
        r0 = jnp.where(c0 == m, lane, BIG)
        r1 = jnp.where(c1 == m, lane + 16, BIG)
        rs = _bmin(jnp.minimum(r0, r1), lane)
        gidx = rs * 16 + ls
        lane0 = lane == 0
        jv = jnp.full((16,), j, jnp.int32)
        plsc.store_scatter(lv_v, [jv], m, mask=lane0)
        plsc.store_scatter(li_v, [jv], gidx + base, mask=lane0)
        plsc.store_scatter(lg_v, [gidx],
                          jnp.full((16,), NEG, jnp.float32), mask=lane0)
        c0 = jnp.where(lane == rs, NEG, c0)
        c1 = jnp.where(lane + 16 == rs, NEG, c1)
        m2 = _bmax(jnp.maximum(c0, c1), lane)
        return jnp.where(lane == ls, m2, mv)

    lax.fori_loop(0, K, extract_one, mv)

    pltpu.sync_copy(lv_v, sv_sh.at[pl.ds(s * K, K)])
    pltpu.sync_copy(li_v, si_sh.at[pl.ds(s * K, K)])
    plsc.subcore_barrier()

    if True:
        pltpu.sync_copy(sv_sh, cv_v)
        pltpu.sync_copy(si_sh, ci_v)

        def merge_body(j, carry):
            ptr, resv0, resv1, resi0, resi1 = carry
            pc = lane * K + jnp.minimum(ptr, K - 1)
            hv = plsc.load_gather(cv_v, [pc])
            hi = plsc.load_gather(ci_v, [pc])
            hv = jnp.where(ptr > K - 1, NEG, hv)
            m = _bmax(hv, lane)
            cand = jnp.where(hv == m, hi, BIG)
            gi = _bmin(cand, lane)
            pick = jnp.logical_and(cand == gi, hv == m)
            ptr = ptr + jnp.where(pick, 1, 0)
            jm = lax.rem(j, 16)
            in0 = j < 16
            at = lane == jm
            resv0 = jnp.where(jnp.logical_and(in0, at), m, resv0)
            resi0 = jnp.where(jnp.logical_and(in0, at), gi, resi0)
            resv1 = jnp.where(jnp.logical_and(jnp.logical_not(in0), at), m, resv1)
            resi1 = jnp.where(jnp.logical_and(jnp.logical_not(in0), at), gi, resi1)
            return ptr, resv0, resv1, resi0, resi1

        init = (jnp.zeros((16,), jnp.int32),
                jnp.full((16,), NEG, jnp.float32),
                jnp.full((16,), NEG, jnp.float32),
                jnp.zeros((16,), jnp.int32),
                jnp.zeros((16,), jnp.int32))
        _, resv0, resv1, resi0, resi1 = lax.fori_loop(0, K, merge_body, init)

        is0 = c == 0
        myv = jnp.where(is0, resv0, resv1)
        myi = jnp.where(is0, resi0, resi1)

        @pl.when(s == 0)
        def _():
            val_v[...] = myv
            pltpu.sync_copy(val_v, val_out.at[pl.ds(c * 16, 16)])

        mine = jnp.take_along_axis(myi, jnp.full((16,), s, jnp.int32), axis=0)
        plsc.store_scatter(idx1_v, [jnp.zeros((16,), jnp.int32)], mine,
                           mask=lane == 0)
        pltpu.async_copy(e_hbm.at[idx1_v], row1_v, sem).wait()
        pltpu.sync_copy(row1_v, sel_out.at[pl.ds(c * 16 + s, 1)])


_topk_gather = functools.partial(
    pl.kernel,
    out_type=(jax.ShapeDtypeStruct((K, D), jnp.float32),
              jax.ShapeDtypeStruct((K,), jnp.float32)),
    mesh=plsc.VectorSubcoreMesh(core_axis_name="c", subcore_axis_name="s"),
    compiler_params=pltpu.CompilerParams(needs_layout_passes=False),
    scratch_types=[
        pltpu.VMEM((CHUNK,), jnp.float32),
        pltpu.VMEM_SHARED((NSUB * K,), jnp.float32),
        pltpu.VMEM_SHARED((NSUB * K,), jnp.int32),
        pltpu.VMEM((NSUB * K,), jnp.float32),
        pltpu.VMEM((NSUB * K,), jnp.int32),
        pltpu.VMEM((K,), jnp.float32),
        pltpu.VMEM((K,), jnp.int32),
        pltpu.VMEM((16,), jnp.int32),
        pltpu.VMEM((16,), jnp.float32),
        pltpu.VMEM((1,), jnp.int32),
        pltpu.VMEM((1, D), jnp.float32),
        pltpu.SemaphoreType.DMA,
    ],
)(_topk_body)


def _finish_body(vm, vz, topv, sel, wv, bv, wo, bo, out):
    m = jnp.max(vm[...])
    z = jnp.sum(vz[...] * jnp.exp(vm[...] - m))
    s = jnp.exp(topv[...].reshape(1, K) - m) / z
    ssum = jnp.sum(s)
    ws = jnp.dot(s, sel[...], preferred_element_type=jnp.float32)
    h = (jnp.dot(ws, wv[...], preferred_element_type=jnp.float32)
         + ssum * bv[...].reshape(1, H))
    o = jnp.dot(h, wo[...], preferred_element_type=jnp.float32)
    out[...] = (o + bo[...].reshape(1, O)).reshape(O)


def _finish_call(vm, vz, topv, sel, Wv, bv, Wo, bo):
    return pl.pallas_call(
        _finish_body,
        out_shape=jax.ShapeDtypeStruct((O,), jnp.float32),
    )(vm, vz, topv, sel, Wv, bv, Wo, bo)


def kernel(class_embedding, entity_embeddings, Wq, bq, Wk, bk, Wv, bv, Wo, bo):
    del bk
    lg, vm, vz = _logits_call(class_embedding, Wq, bq, Wk, entity_embeddings)
    sel, topv = _topk_gather(lg, entity_embeddings)
    return _finish_call(vm, vz, topv, sel, Wv, bv, Wo, bo)
